# baseline (speedup 1.0000x reference)
import functools

import jax
import jax.numpy as jnp
import numpy as np
from jax.experimental import pallas as pl
from jax.experimental.pallas import tpu as pltpu


def _gelu(v):
    return 0.5 * v * (1.0 + jax.lax.erf(v * np.float32(0.7071067811865476)))


def _fused_body(x_ref, gw1_ref, gw2_ref, gb1_ref, gb2_ref, eb_ref,
                p2w_ref, p4w_ref, ru_ref, rb_ref, fdw_ref, fdb_ref, fuw_ref,
                fub_ref,
                ra_ref, rbias_ref, p2v_ref, p4v_ref, p2b_ref, p4b_ref,
                pbias2_ref, pbias4_ref, sel_ref,
                p2a_ref, p4a_ref, alpha_ref,
                o_ref, w1s, gw2s, b2s, w2s, *, E, H, R, NL, K1P, K2):
    bf16 = jnp.bfloat16
    f32 = jnp.float32
    i = pl.program_id(0)

    @pl.when(i == 0)
    def _pack():
        alpha = alpha_ref[0, 0]
        cat16 = jnp.concatenate([
            p2w_ref[...], p4w_ref[...], ru_ref[...],
            jnp.zeros((2, E), f32),
        ], axis=0)
        w1s[...] = jnp.concatenate([
            gw1_ref[...],
            fdw_ref[0:E, :], fdw_ref[E:2 * E, :],
            cat16.T,
            jnp.zeros((E, K1P - H - 2 * R - 16), f32),
        ], axis=1).astype(bf16)
        gw2s[...] = gw2_ref[...].astype(bf16)
        b2s[...] = jnp.concatenate([
            fdb_ref[0:1, :], fdb_ref[1:2, :],
            p2b_ref[0:1, :], p2b_ref[1:2, :],
            p4b_ref[0:1, :], p4b_ref[1:2, :],
            rb_ref[...],
        ], axis=1)
        zE1 = jnp.zeros((1, E), f32)
        pv_rows = (
            [p2v_ref[j:j + 1, :] * p2a_ref[j // 2, j % 2] for j in range(4)]
            + [p4v_ref[j:j + 1, :] * p4a_ref[j // 4, j % 4] for j in range(8)]
        )
        w2s[...] = (jnp.concatenate([
            fuw_ref[...],
            jnp.concatenate(
                [jnp.zeros((12, E), f32),
                 jnp.concatenate(pv_rows, axis=0)], axis=1),
            jnp.concatenate([ra_ref[...], jnp.zeros((2, E), f32)], axis=1),
            jnp.concatenate([zE1, rbias_ref[0:1, :]], axis=1),
            fub_ref[0:1, :],
            jnp.concatenate([zE1, pbias2_ref[0:1, :]], axis=1),
            jnp.concatenate([zE1, pbias4_ref[0:1, :]], axis=1),
            jnp.concatenate([zE1, rbias_ref[1:2, :]], axis=1),
            fub_ref[1:2, :],
            jnp.concatenate([zE1, pbias2_ref[1:2, :]], axis=1),
            jnp.concatenate([zE1, pbias4_ref[1:2, :]], axis=1),
        ], axis=0) * alpha).astype(bf16)

    xt = x_ref[...]
    y = jnp.dot(xt.astype(bf16), w1s[...], preferred_element_type=f32)
    h = _gelu(y[:, :H] + gb1_ref[...])
    logits = (jnp.dot(h.astype(bf16), gw2s[...], preferred_element_type=f32)
              + (gb2_ref[...] + eb_ref[...]))
    probs = jax.nn.softmax(logits, axis=-1)
    nlp = y[:, H:H + NL] + b2s[...]
    nl = jnp.concatenate(
        [_gelu(nlp[:, :NL - 2]), jax.nn.sigmoid(nlp[:, NL - 2:])],
        axis=1)
    scale = jnp.dot(probs, sel_ref[...], preferred_element_type=f32)
    z = jnp.concatenate([nl * scale, probs], axis=1)
    o = jnp.dot(z.astype(bf16), w2s[...], preferred_element_type=f32)
    o_ref[...] = xt * (1.0 + o[:, :E]) + o[:, E:]


def kernel(x, reglu_u, reglu_a, reglu_b, reglu_bias, film_dw, film_db,
           film_uw, film_ub, p2_w, p2_v, p2_alpha, p2_b, p2_bias, p4_w, p4_v,
           p4_alpha, p4_b, p4_bias, gate_w1, gate_b1, gate_w2, gate_b2,
           expert_bias, post_mix_alpha):
    B, S, E = x.shape
    H = gate_w1.shape[1]
    R = film_dw.shape[-1]
    r2 = p2_w.shape[1]
    r4 = p4_w.shape[1]
    M = gate_w2.shape[1]
    NL = 2 * R + 2 * r2 + 2 * r4 + 2
    K1P = 384
    K2 = NL + M

    f32 = jnp.float32
    bf16 = jnp.bfloat16

    sel_np = np.zeros((M, NL), dtype=np.float32)
    c = 0
    sel_np[1, c:c + R] = 1.0; c += R
    sel_np[5, c:c + R] = 1.0; c += R
    sel_np[2, c:c + r2] = 1.0; c += r2
    sel_np[6, c:c + r2] = 1.0; c += r2
    sel_np[3, c:c + r4] = 1.0; c += r4
    sel_np[7, c:c + r4] = 1.0; c += r4
    sel_np[0, c] = 1.0; c += 1
    sel_np[4, c] = 1.0; c += 1
    sel = jnp.asarray(sel_np)

    N = B * S
    T = 1024
    x2 = x.reshape(N, E)

    def full(shape):
        n = len(shape)
        return pl.BlockSpec(shape, lambda i, _n=n: (0,) * _n)

    smem = pl.BlockSpec(memory_space=pltpu.SMEM)
    body = functools.partial(_fused_body, E=E, H=H, R=R, NL=NL, K1P=K1P,
                             K2=K2)
    out = pl.pallas_call(
        body,
        grid=(N // T,),
        in_specs=[
            pl.BlockSpec((T, E), lambda i: (i, 0)),
            full((E, H)),
            full((H, M)),
            full((1, H)),
            full((1, M)),
            full((1, M)),
            full((2 * r2, E)),
            full((2 * r4, E)),
            full((2, E)),
            full((1, 2)),
            full((2 * E, R)),
            full((2, R)),
            full((2 * R, 2 * E)),
            full((2, 2 * E)),
            full((2, E)),
            full((2, E)),
            full((2 * r2, E)),
            full((2 * r4, E)),
            full((2, r2)),
            full((2, r4)),
            full((2, E)),
            full((2, E)),
            full((M, NL)),
            smem,
            smem,
            smem,
        ],
        out_specs=pl.BlockSpec((T, E), lambda i: (i, 0)),
        out_shape=jax.ShapeDtypeStruct((N, E), f32),
        scratch_shapes=[
            pltpu.VMEM((E, K1P), bf16),
            pltpu.VMEM((H, M), bf16),
            pltpu.VMEM((1, NL), f32),
            pltpu.VMEM((K2, 2 * E), bf16),
        ],
    )(x2, gate_w1, gate_w2, gate_b1[None, :], gate_b2[None, :],
      expert_bias[None, :], p2_w.reshape(2 * r2, E), p4_w.reshape(2 * r4, E),
      reglu_u, reglu_b[None, :], film_dw.reshape(2 * E, R), film_db,
      film_uw.reshape(2 * R, 2 * E), film_ub, reglu_a, reglu_bias,
      p2_v.reshape(2 * r2, E), p4_v.reshape(2 * r4, E), p2_b, p4_b,
      p2_bias, p4_bias, sel, p2_alpha, p4_alpha, post_mix_alpha.reshape(1, 1))
    return out.reshape(B, S, E)


# speedup vs baseline: 1.1587x; 1.1587x over previous
import functools

import jax
import jax.numpy as jnp
import numpy as np
from jax.experimental import pallas as pl
from jax.experimental.pallas import tpu as pltpu


def _gelu(v):
    return 0.5 * v * (1.0 + jax.lax.erf(v * np.float32(0.7071067811865476)))


def _fused_body(x_ref, gw1_ref, gw2_ref, gb1_ref, gb2_ref, eb_ref,
                p2w_ref, p4w_ref, ru_ref, rb_ref, fdw_ref, fdb_ref, fuw_ref,
                fub_ref,
                ra_ref, rbias_ref, p2v_ref, p4v_ref, p2b_ref, p4b_ref,
                pbias2_ref, pbias4_ref, sel_ref,
                p2a_ref, p4a_ref, alpha_ref,
                o_ref, w1s, gw2s, b2s, w2s, *, E, H, R, NL, K1P, K2):
    bf16 = jnp.bfloat16
    f32 = jnp.float32
    i = pl.program_id(0)

    @pl.when(i == 0)
    def _pack():
        alpha = alpha_ref[0, 0]
        cat16 = jnp.concatenate([
            p2w_ref[0], p2w_ref[1], p4w_ref[0], p4w_ref[1], ru_ref[...],
            jnp.zeros((2, E), f32),
        ], axis=0)
        w1s[...] = jnp.concatenate([
            gw1_ref[...],
            fdw_ref[0], fdw_ref[1],
            cat16.T,
            jnp.zeros((E, K1P - H - 2 * R - 16), f32),
        ], axis=1).astype(bf16)
        gw2s[...] = gw2_ref[...].astype(bf16)
        b2s[...] = jnp.concatenate([
            fdb_ref[0:1, :], fdb_ref[1:2, :],
            p2b_ref[0:1, :], p2b_ref[1:2, :],
            p4b_ref[0:1, :], p4b_ref[1:2, :],
            rb_ref[...][None, :],
        ], axis=1)
        zE1 = jnp.zeros((1, E), f32)
        pv_rows = (
            [p2v_ref[j // 2, j % 2:j % 2 + 1, :] * p2a_ref[j // 2, j % 2]
             for j in range(4)]
            + [p4v_ref[j // 4, j % 4:j % 4 + 1, :] * p4a_ref[j // 4, j % 4]
               for j in range(8)]
        )
        w2s[...] = (jnp.concatenate([
            fuw_ref[0], fuw_ref[1],
            jnp.concatenate(
                [jnp.zeros((12, E), f32),
                 jnp.concatenate(pv_rows, axis=0)], axis=1),
            jnp.concatenate([ra_ref[...], jnp.zeros((2, E), f32)], axis=1),
            jnp.concatenate([zE1, rbias_ref[0:1, :]], axis=1),
            fub_ref[0:1, :],
            jnp.concatenate([zE1, pbias2_ref[0:1, :]], axis=1),
            jnp.concatenate([zE1, pbias4_ref[0:1, :]], axis=1),
            jnp.concatenate([zE1, rbias_ref[1:2, :]], axis=1),
            fub_ref[1:2, :],
            jnp.concatenate([zE1, pbias2_ref[1:2, :]], axis=1),
            jnp.concatenate([zE1, pbias4_ref[1:2, :]], axis=1),
        ], axis=0) * alpha).astype(bf16)

    xt = x_ref[...]
    y = jnp.dot(xt.astype(bf16), w1s[...], preferred_element_type=f32)
    h = _gelu(y[:, :H] + gb1_ref[...][None, :])
    logits = (jnp.dot(h.astype(bf16), gw2s[...], preferred_element_type=f32)
              + (gb2_ref[...] + eb_ref[...])[None, :])
    probs = jax.nn.softmax(logits, axis=-1)
    nlp = y[:, H:H + NL] + b2s[...]
    nl = jnp.concatenate(
        [_gelu(nlp[:, :NL - 2]), jax.nn.sigmoid(nlp[:, NL - 2:])],
        axis=1)
    scale = jnp.dot(probs, sel_ref[...], preferred_element_type=f32)
    z = jnp.concatenate([nl * scale, probs], axis=1)
    o = jnp.dot(z.astype(bf16), w2s[...], preferred_element_type=f32)
    o_ref[...] = xt * (1.0 + o[:, :E]) + o[:, E:]


def kernel(x, reglu_u, reglu_a, reglu_b, reglu_bias, film_dw, film_db,
           film_uw, film_ub, p2_w, p2_v, p2_alpha, p2_b, p2_bias, p4_w, p4_v,
           p4_alpha, p4_b, p4_bias, gate_w1, gate_b1, gate_w2, gate_b2,
           expert_bias, post_mix_alpha):
    B, S, E = x.shape
    H = gate_w1.shape[1]
    R = film_dw.shape[-1]
    r2 = p2_w.shape[1]
    r4 = p4_w.shape[1]
    M = gate_w2.shape[1]
    NL = 2 * R + 2 * r2 + 2 * r4 + 2
    K1P = 384
    K2 = NL + M

    f32 = jnp.float32
    bf16 = jnp.bfloat16

    sel_np = np.zeros((M, NL), dtype=np.float32)
    c = 0
    sel_np[1, c:c + R] = 1.0; c += R
    sel_np[5, c:c + R] = 1.0; c += R
    sel_np[2, c:c + r2] = 1.0; c += r2
    sel_np[6, c:c + r2] = 1.0; c += r2
    sel_np[3, c:c + r4] = 1.0; c += r4
    sel_np[7, c:c + r4] = 1.0; c += r4
    sel_np[0, c] = 1.0; c += 1
    sel_np[4, c] = 1.0; c += 1
    sel = jnp.asarray(sel_np)

    N = B * S
    T = 2048
    x2 = x.reshape(N, E)

    def full(shape):
        n = len(shape)
        return pl.BlockSpec(shape, lambda i, _n=n: (0,) * _n)

    smem = pl.BlockSpec(memory_space=pltpu.SMEM)
    body = functools.partial(_fused_body, E=E, H=H, R=R, NL=NL, K1P=K1P,
                             K2=K2)
    out = pl.pallas_call(
        body,
        grid=(N // T,),
        in_specs=[
            pl.BlockSpec((T, E), lambda i: (i, 0)),
            full((E, H)),
            full((H, M)),
            full((H,)),
            full((M,)),
            full((M,)),
            full((2, r2, E)),
            full((2, r4, E)),
            full((2, E)),
            full((2,)),
            full((2, E, R)),
            full((2, R)),
            full((2, R, 2 * E)),
            full((2, 2 * E)),
            full((2, E)),
            full((2, E)),
            full((2, r2, E)),
            full((2, r4, E)),
            full((2, r2)),
            full((2, r4)),
            full((2, E)),
            full((2, E)),
            full((M, NL)),
            smem,
            smem,
            smem,
        ],
        out_specs=pl.BlockSpec((T, E), lambda i: (i, 0)),
        out_shape=jax.ShapeDtypeStruct((N, E), f32),
        scratch_shapes=[
            pltpu.VMEM((E, K1P), bf16),
            pltpu.VMEM((H, M), bf16),
            pltpu.VMEM((1, NL), f32),
            pltpu.VMEM((K2, 2 * E), bf16),
        ],
    )(x2, gate_w1, gate_w2, gate_b1, gate_b2,
      expert_bias, p2_w, p4_w,
      reglu_u, reglu_b, film_dw, film_db,
      film_uw, film_ub, reglu_a, reglu_bias,
      p2_v, p4_v, p2_b, p4_b,
      p2_bias, p4_bias, sel, p2_alpha, p4_alpha, post_mix_alpha.reshape(1, 1))
    return out.reshape(B, S, E)


# speedup vs baseline: 1.1629x; 1.0036x over previous
import functools

import jax
import jax.numpy as jnp
import numpy as np
from jax.experimental import pallas as pl
from jax.experimental.pallas import tpu as pltpu


def _gelu(v):
    return 0.5 * v * (1.0 + jax.lax.erf(v * np.float32(0.7071067811865476)))


def _fused_body(x_ref, gw1_ref, gw2_ref, gb1_ref, gb2_ref, eb_ref,
                p2w_ref, p4w_ref, ru_ref, rb_ref, fdw_ref, fdb_ref, fuw_ref,
                fub_ref,
                ra_ref, rbias_ref, p2v_ref, p4v_ref, p2b_ref, p4b_ref,
                pbias2_ref, pbias4_ref, sel_ref,
                p2a_ref, p4a_ref, alpha_ref,
                o_ref, w1s, gw2s, b2s, w2s, *, E, H, R, NL, K1P, K2):
    bf16 = jnp.bfloat16
    f32 = jnp.float32
    i = pl.program_id(0)

    @pl.when(i == 0)
    def _pack():
        alpha = alpha_ref[0, 0]
        cat16 = jnp.concatenate([
            p2w_ref[0], p2w_ref[1], p4w_ref[0], p4w_ref[1], ru_ref[...],
            jnp.zeros((2, E), f32),
        ], axis=0)
        w1s[...] = jnp.concatenate([
            gw1_ref[...],
            fdw_ref[0], fdw_ref[1],
            cat16.T,
            jnp.zeros((E, K1P - H - 2 * R - 16), f32),
        ], axis=1).astype(bf16)
        gw2s[...] = gw2_ref[...].astype(bf16)
        b2s[...] = jnp.concatenate([
            fdb_ref[0:1, :], fdb_ref[1:2, :],
            p2b_ref[0:1, :], p2b_ref[1:2, :],
            p4b_ref[0:1, :], p4b_ref[1:2, :],
            rb_ref[...][None, :],
        ], axis=1)
        zE1 = jnp.zeros((1, E), f32)
        pv_rows = (
            [p2v_ref[j // 2, j % 2:j % 2 + 1, :] * p2a_ref[j // 2, j % 2]
             for j in range(4)]
            + [p4v_ref[j // 4, j % 4:j % 4 + 1, :] * p4a_ref[j // 4, j % 4]
               for j in range(8)]
        )
        w2s[...] = (jnp.concatenate([
            fuw_ref[0], fuw_ref[1],
            jnp.concatenate(
                [jnp.zeros((12, E), f32),
                 jnp.concatenate(pv_rows, axis=0)], axis=1),
            jnp.concatenate([ra_ref[...], jnp.zeros((2, E), f32)], axis=1),
            jnp.concatenate([zE1, rbias_ref[0:1, :]], axis=1),
            fub_ref[0:1, :],
            jnp.concatenate([zE1, pbias2_ref[0:1, :]], axis=1),
            jnp.concatenate([zE1, pbias4_ref[0:1, :]], axis=1),
            jnp.concatenate([zE1, rbias_ref[1:2, :]], axis=1),
            fub_ref[1:2, :],
            jnp.concatenate([zE1, pbias2_ref[1:2, :]], axis=1),
            jnp.concatenate([zE1, pbias4_ref[1:2, :]], axis=1),
        ], axis=0) * alpha).astype(bf16)

    xt = x_ref[...]
    y = jnp.dot(xt.astype(bf16), w1s[...], preferred_element_type=f32)
    h = _gelu(y[:, :H] + gb1_ref[...][None, :])
    logits = (jnp.dot(h.astype(bf16), gw2s[...], preferred_element_type=f32)
              + (gb2_ref[...] + eb_ref[...])[None, :])
    m = jnp.max(logits, axis=-1, keepdims=True)
    eexp = jnp.exp(logits - m)
    probs = eexp * (1.0 / jnp.sum(eexp, axis=-1, keepdims=True))
    nlp = y[:, H:H + NL] + b2s[...]
    nl = jnp.concatenate(
        [_gelu(nlp[:, :NL - 2]), jax.nn.sigmoid(nlp[:, NL - 2:])],
        axis=1)
    scale = jnp.dot(probs, sel_ref[...], preferred_element_type=f32)
    z = jnp.concatenate([nl * scale, probs], axis=1).astype(bf16)
    oc = jnp.dot(z, w2s[:, :E], preferred_element_type=f32)
    oa = jnp.dot(z, w2s[:, E:], preferred_element_type=f32)
    o_ref[...] = xt * (1.0 + oc) + oa


def kernel(x, reglu_u, reglu_a, reglu_b, reglu_bias, film_dw, film_db,
           film_uw, film_ub, p2_w, p2_v, p2_alpha, p2_b, p2_bias, p4_w, p4_v,
           p4_alpha, p4_b, p4_bias, gate_w1, gate_b1, gate_w2, gate_b2,
           expert_bias, post_mix_alpha):
    B, S, E = x.shape
    H = gate_w1.shape[1]
    R = film_dw.shape[-1]
    r2 = p2_w.shape[1]
    r4 = p4_w.shape[1]
    M = gate_w2.shape[1]
    NL = 2 * R + 2 * r2 + 2 * r4 + 2
    K1P = 384
    K2 = NL + M

    f32 = jnp.float32
    bf16 = jnp.bfloat16

    sel_np = np.zeros((M, NL), dtype=np.float32)
    c = 0
    sel_np[1, c:c + R] = 1.0; c += R
    sel_np[5, c:c + R] = 1.0; c += R
    sel_np[2, c:c + r2] = 1.0; c += r2
    sel_np[6, c:c + r2] = 1.0; c += r2
    sel_np[3, c:c + r4] = 1.0; c += r4
    sel_np[7, c:c + r4] = 1.0; c += r4
    sel_np[0, c] = 1.0; c += 1
    sel_np[4, c] = 1.0; c += 1
    sel = jnp.asarray(sel_np)

    N = B * S
    T = 2048
    x2 = x.reshape(N, E)

    def full(shape):
        n = len(shape)
        return pl.BlockSpec(shape, lambda i, _n=n: (0,) * _n)

    smem = pl.BlockSpec(memory_space=pltpu.SMEM)
    body = functools.partial(_fused_body, E=E, H=H, R=R, NL=NL, K1P=K1P,
                             K2=K2)
    out = pl.pallas_call(
        body,
        grid=(N // T,),
        in_specs=[
            pl.BlockSpec((T, E), lambda i: (i, 0)),
            full((E, H)),
            full((H, M)),
            full((H,)),
            full((M,)),
            full((M,)),
            full((2, r2, E)),
            full((2, r4, E)),
            full((2, E)),
            full((2,)),
            full((2, E, R)),
            full((2, R)),
            full((2, R, 2 * E)),
            full((2, 2 * E)),
            full((2, E)),
            full((2, E)),
            full((2, r2, E)),
            full((2, r4, E)),
            full((2, r2)),
            full((2, r4)),
            full((2, E)),
            full((2, E)),
            full((M, NL)),
            smem,
            smem,
            smem,
        ],
        out_specs=pl.BlockSpec((T, E), lambda i: (i, 0)),
        out_shape=jax.ShapeDtypeStruct((N, E), f32),
        scratch_shapes=[
            pltpu.VMEM((E, K1P), bf16),
            pltpu.VMEM((H, M), bf16),
            pltpu.VMEM((1, NL), f32),
            pltpu.VMEM((K2, 2 * E), bf16),
        ],
    )(x2, gate_w1, gate_w2, gate_b1, gate_b2,
      expert_bias, p2_w, p4_w,
      reglu_u, reglu_b, film_dw, film_db,
      film_uw, film_ub, reglu_a, reglu_bias,
      p2_v, p4_v, p2_b, p4_b,
      p2_bias, p4_bias, sel, p2_alpha, p4_alpha, post_mix_alpha.reshape(1, 1))
    return out.reshape(B, S, E)
